# baseline (speedup 1.0000x reference)
import functools

import jax
import jax.numpy as jnp
from jax import lax
from jax.experimental import pallas as pl
from jax.experimental.pallas import tpu as pltpu
from jax.experimental.pallas import tpu_sc as plsc

N = 10000
D = 128
DC = 64
C = 40
NSC = 2
NTILES = 16
EDGE_BLOCKS = NSC * NTILES
CHUNK = 112
CHUNKS_PER_TILE = 92
E_PAD = EDGE_BLOCKS * CHUNKS_PER_TILE * CHUNK
ACC_ROWS = N + 16
ZROWS = ACC_ROWS // NTILES
OROWS = N // NTILES


def _make_msgpass(d):
  mesh = plsc.VectorSubcoreMesh(core_axis_name="c", subcore_axis_name="s")

  @functools.partial(
      pl.kernel,
      out_type=jax.ShapeDtypeStruct((NSC, N, d), jnp.float32),
      mesh=mesh,
      scratch_types=[
          pltpu.VMEM((CHUNKS_PER_TILE, CHUNK), jnp.int32),
          pltpu.VMEM((CHUNKS_PER_TILE, CHUNK), jnp.int32),
          pltpu.VMEM((CHUNK, d), jnp.float32),
          pltpu.VMEM((CHUNK, d), jnp.float32),
          pltpu.VMEM_SHARED((ACC_ROWS, d), jnp.float32),
          pltpu.SemaphoreType.DMA,
          pltpu.SemaphoreType.DMA,
          pltpu.SemaphoreType.DMA,
          pltpu.SemaphoreType.DMA,
      ],
      compiler_params=pltpu.CompilerParams(use_tc_tiling_on_sc=False),
  )
  def msgpass(src_hbm, dst_hbm, zeros_hbm, y_hbm, out_hbm,
              idx_s, idx_d, rows0, rows1, acc, gsem0, gsem1, ssem0, ssem1):
    c = lax.axis_index("c")
    s = lax.axis_index("s")
    blk = c * NTILES + s
    pltpu.sync_copy(src_hbm.at[blk], idx_s)
    pltpu.sync_copy(dst_hbm.at[blk], idx_d)
    pltpu.sync_copy(zeros_hbm, acc.at[pl.ds(s * ZROWS, ZROWS)])
    plsc.subcore_barrier()

    pltpu.async_copy(y_hbm.at[idx_s.at[0]], rows0, gsem0)
    pltpu.async_copy(y_hbm.at[idx_s.at[1]], rows1, gsem1)

    @pl.loop(0, CHUNKS_PER_TILE, step=2)
    def _(j0):
      pltpu.make_async_copy(y_hbm.at[idx_s.at[j0]], rows0, gsem0).wait()
      pltpu.async_copy(rows0, acc.at[idx_d.at[j0]], ssem0, add=True)
      pltpu.make_async_copy(y_hbm.at[idx_s.at[j0 + 1]], rows1, gsem1).wait()
      pltpu.async_copy(rows1, acc.at[idx_d.at[j0 + 1]], ssem1, add=True)
      pltpu.make_async_copy(rows0, acc.at[idx_d.at[j0]], ssem0).wait()

      @pl.when(j0 + 2 < CHUNKS_PER_TILE)
      def _():
        pltpu.async_copy(y_hbm.at[idx_s.at[j0 + 2]], rows0, gsem0)

      pltpu.make_async_copy(rows1, acc.at[idx_d.at[j0 + 1]], ssem1).wait()

      @pl.when(j0 + 3 < CHUNKS_PER_TILE)
      def _():
        pltpu.async_copy(y_hbm.at[idx_s.at[j0 + 3]], rows1, gsem1)

    plsc.subcore_barrier()
    pltpu.sync_copy(acc.at[pl.ds(s * OROWS, OROWS)],
                    out_hbm.at[c, pl.ds(s * OROWS, OROWS)])

  return msgpass


_R = 1000


def _tc_fuse1(a0p, W0, b0, Wi0, bi0):
  def body(a_ref, w_ref, b_ref, wi_ref, bi_ref, h_ref, ie_ref):
    a = a_ref[0] + a_ref[1]
    h = jnp.maximum(
        jnp.dot(a, w_ref[...], preferred_element_type=jnp.float32) + b_ref[...],
        0.0)
    h_ref[...] = h
    ie_ref[...] = (
        jnp.dot(h, wi_ref[...], preferred_element_type=jnp.float32) + bi_ref[...])

  return pl.pallas_call(
      body,
      grid=(N // _R,),
      in_specs=[
          pl.BlockSpec((NSC, _R, D), lambda i: (0, i, 0)),
          pl.BlockSpec((D, D), lambda i: (0, 0)),
          pl.BlockSpec((1, D), lambda i: (0, 0)),
          pl.BlockSpec((D, C), lambda i: (0, 0)),
          pl.BlockSpec((1, C), lambda i: (0, 0)),
      ],
      out_specs=[
          pl.BlockSpec((_R, D), lambda i: (i, 0)),
          pl.BlockSpec((_R, C), lambda i: (i, 0)),
      ],
      out_shape=[
          jax.ShapeDtypeStruct((N, D), jnp.float32),
          jax.ShapeDtypeStruct((N, C), jnp.float32),
      ],
  )(a0p, W0, b0, Wi0, bi0)


def _tc_fuse2(a1p, W1, b1, Wi1, bi1, W2p, b2, ie0):
  def body(a_ref, w_ref, b_ref, wi_ref, bi_ref, w2_ref, b2_ref, ie_ref,
           ie2_ref, y2_ref):
    a = a_ref[0] + a_ref[1]
    h = jnp.maximum(
        jnp.dot(a, w_ref[...], preferred_element_type=jnp.float32) + b_ref[...],
        0.0)
    ie2_ref[...] = (
        ie_ref[...]
        + jnp.dot(h, wi_ref[...], preferred_element_type=jnp.float32)
        + bi_ref[...] + b2_ref[...])
    y2_ref[...] = jnp.dot(h, w2_ref[...], preferred_element_type=jnp.float32)

  return pl.pallas_call(
      body,
      grid=(N // _R,),
      in_specs=[
          pl.BlockSpec((NSC, _R, D), lambda i: (0, i, 0)),
          pl.BlockSpec((D, D), lambda i: (0, 0)),
          pl.BlockSpec((1, D), lambda i: (0, 0)),
          pl.BlockSpec((D, C), lambda i: (0, 0)),
          pl.BlockSpec((1, C), lambda i: (0, 0)),
          pl.BlockSpec((D, DC), lambda i: (0, 0)),
          pl.BlockSpec((1, C), lambda i: (0, 0)),
          pl.BlockSpec((_R, C), lambda i: (i, 0)),
      ],
      out_specs=[
          pl.BlockSpec((_R, C), lambda i: (i, 0)),
          pl.BlockSpec((_R, DC), lambda i: (i, 0)),
      ],
      out_shape=[
          jax.ShapeDtypeStruct((N, C), jnp.float32),
          jax.ShapeDtypeStruct((N, DC), jnp.float32),
      ],
  )(a1p, W1, b1, Wi1, bi1, W2p, b2, ie0)


def _tc_fuse3(a2p, ie2):
  def body(a_ref, ie_ref, o_ref):
    o_ref[...] = a_ref[0][:, :C] + a_ref[1][:, :C] + ie_ref[...]

  return pl.pallas_call(
      body,
      grid=(N // _R,),
      in_specs=[
          pl.BlockSpec((NSC, _R, DC), lambda i: (0, i, 0)),
          pl.BlockSpec((_R, C), lambda i: (i, 0)),
      ],
      out_specs=pl.BlockSpec((_R, C), lambda i: (i, 0)),
      out_shape=jax.ShapeDtypeStruct((N, C), jnp.float32),
  )(a2p, ie2)


def kernel(graph, features, W0, b0, W1, b1, W2, b2, Wi0, bi0, Wi1, bi1):
  src, dst = graph[0], graph[1]
  pad = E_PAD - src.shape[0]
  srcp = jnp.concatenate([src, jnp.zeros((pad,), jnp.int32)]).reshape(
      EDGE_BLOCKS, CHUNKS_PER_TILE, CHUNK)
  dstp = jnp.concatenate([dst, jnp.full((pad,), N, jnp.int32)]).reshape(
      EDGE_BLOCKS, CHUNKS_PER_TILE, CHUNK)
  zeros128 = jnp.zeros((ZROWS, D), jnp.float32)
  zeros64 = jnp.zeros((ZROWS, DC), jnp.float32)

  mp128 = _make_msgpass(D)
  mp64 = _make_msgpass(DC)

  a0p = mp128(srcp, dstp, zeros128, features)
  h1, ie0 = _tc_fuse1(a0p, W0, b0.reshape(1, D), Wi0, bi0.reshape(1, C))
  a1p = mp128(srcp, dstp, zeros128, h1)
  W2p = jnp.pad(W2, ((0, 0), (0, DC - C)))
  ie2, y2 = _tc_fuse2(a1p, W1, b1.reshape(1, D), Wi1, bi1.reshape(1, C),
                      W2p, b2.reshape(1, C), ie0)
  a2p = mp64(srcp, dstp, zeros64, y2)
  return _tc_fuse3(a2p, ie2)


# speedup vs baseline: 1.7860x; 1.7860x over previous
import functools

import jax
import jax.numpy as jnp
from jax import lax
from jax.experimental import pallas as pl
from jax.experimental.pallas import tpu as pltpu
from jax.experimental.pallas import tpu_sc as plsc

N = 10000
D = 128
DC = 64
C = 40
NSC = 2
NTILES = 16
EDGE_BLOCKS = NSC * NTILES
CHUNK = 224
CHUNKS_PER_TILE = 45
E_PAD = EDGE_BLOCKS * CHUNKS_PER_TILE * CHUNK
ACC_ROWS = N + 16
ZROWS = ACC_ROWS // NTILES
OROWS = N // NTILES


def _make_msgpass(d):
  mesh = plsc.VectorSubcoreMesh(core_axis_name="c", subcore_axis_name="s")

  @functools.partial(
      pl.kernel,
      out_type=jax.ShapeDtypeStruct((NSC, N, d), jnp.float32),
      mesh=mesh,
      scratch_types=[
          pltpu.VMEM((CHUNKS_PER_TILE, CHUNK), jnp.int32),
          pltpu.VMEM((CHUNKS_PER_TILE, CHUNK), jnp.int32),
          pltpu.VMEM((CHUNK, d), jnp.float32),
          pltpu.VMEM_SHARED((ACC_ROWS, d), jnp.float32),
          pltpu.SemaphoreType.DMA,
      ],
      compiler_params=pltpu.CompilerParams(use_tc_tiling_on_sc=False),
  )
  def msgpass(src_hbm, dst_hbm, zeros_hbm, y_hbm, out_hbm,
              idx_s, idx_d, rows, acc, sem):
    c = lax.axis_index("c")
    s = lax.axis_index("s")
    blk = c * NTILES + s
    pltpu.sync_copy(src_hbm.at[blk], idx_s)
    pltpu.sync_copy(dst_hbm.at[blk], idx_d)
    pltpu.sync_copy(zeros_hbm, acc.at[pl.ds(s * ZROWS, ZROWS)])
    plsc.subcore_barrier()

    @pl.loop(0, CHUNKS_PER_TILE)
    def _(j):
      pltpu.async_copy(y_hbm.at[idx_s.at[j]], rows, sem).wait()
      pltpu.sync_copy(rows, acc.at[idx_d.at[j]], add=True)

    plsc.subcore_barrier()
    pltpu.sync_copy(acc.at[pl.ds(s * OROWS, OROWS)],
                    out_hbm.at[c, pl.ds(s * OROWS, OROWS)])

  return msgpass


_R = 1000


def _tc_fuse1(a0p, W0, b0, Wi0, bi0):
  def body(a_ref, w_ref, b_ref, wi_ref, bi_ref, h_ref, ie_ref):
    a = a_ref[0] + a_ref[1]
    h = jnp.maximum(
        jnp.dot(a, w_ref[...], preferred_element_type=jnp.float32) + b_ref[...],
        0.0)
    h_ref[...] = h
    ie_ref[...] = (
        jnp.dot(h, wi_ref[...], preferred_element_type=jnp.float32) + bi_ref[...])

  return pl.pallas_call(
      body,
      grid=(N // _R,),
      in_specs=[
          pl.BlockSpec((NSC, _R, D), lambda i: (0, i, 0)),
          pl.BlockSpec((D, D), lambda i: (0, 0)),
          pl.BlockSpec((1, D), lambda i: (0, 0)),
          pl.BlockSpec((D, C), lambda i: (0, 0)),
          pl.BlockSpec((1, C), lambda i: (0, 0)),
      ],
      out_specs=[
          pl.BlockSpec((_R, D), lambda i: (i, 0)),
          pl.BlockSpec((_R, C), lambda i: (i, 0)),
      ],
      out_shape=[
          jax.ShapeDtypeStruct((N, D), jnp.float32),
          jax.ShapeDtypeStruct((N, C), jnp.float32),
      ],
  )(a0p, W0, b0, Wi0, bi0)


def _tc_fuse2(a1p, W1, b1, Wi1, bi1, W2p, b2, ie0):
  def body(a_ref, w_ref, b_ref, wi_ref, bi_ref, w2_ref, b2_ref, ie_ref,
           ie2_ref, y2_ref):
    a = a_ref[0] + a_ref[1]
    h = jnp.maximum(
        jnp.dot(a, w_ref[...], preferred_element_type=jnp.float32) + b_ref[...],
        0.0)
    ie2_ref[...] = (
        ie_ref[...]
        + jnp.dot(h, wi_ref[...], preferred_element_type=jnp.float32)
        + bi_ref[...] + b2_ref[...])
    y2_ref[...] = jnp.dot(h, w2_ref[...], preferred_element_type=jnp.float32)

  return pl.pallas_call(
      body,
      grid=(N // _R,),
      in_specs=[
          pl.BlockSpec((NSC, _R, D), lambda i: (0, i, 0)),
          pl.BlockSpec((D, D), lambda i: (0, 0)),
          pl.BlockSpec((1, D), lambda i: (0, 0)),
          pl.BlockSpec((D, C), lambda i: (0, 0)),
          pl.BlockSpec((1, C), lambda i: (0, 0)),
          pl.BlockSpec((D, DC), lambda i: (0, 0)),
          pl.BlockSpec((1, C), lambda i: (0, 0)),
          pl.BlockSpec((_R, C), lambda i: (i, 0)),
      ],
      out_specs=[
          pl.BlockSpec((_R, C), lambda i: (i, 0)),
          pl.BlockSpec((_R, DC), lambda i: (i, 0)),
      ],
      out_shape=[
          jax.ShapeDtypeStruct((N, C), jnp.float32),
          jax.ShapeDtypeStruct((N, DC), jnp.float32),
      ],
  )(a1p, W1, b1, Wi1, bi1, W2p, b2, ie0)


def _tc_fuse3(a2p, ie2):
  def body(a_ref, ie_ref, o_ref):
    o_ref[...] = a_ref[0][:, :C] + a_ref[1][:, :C] + ie_ref[...]

  return pl.pallas_call(
      body,
      grid=(N // _R,),
      in_specs=[
          pl.BlockSpec((NSC, _R, DC), lambda i: (0, i, 0)),
          pl.BlockSpec((_R, C), lambda i: (i, 0)),
      ],
      out_specs=pl.BlockSpec((_R, C), lambda i: (i, 0)),
      out_shape=jax.ShapeDtypeStruct((N, C), jnp.float32),
  )(a2p, ie2)


def kernel(graph, features, W0, b0, W1, b1, W2, b2, Wi0, bi0, Wi1, bi1):
  src, dst = graph[0], graph[1]
  pad = E_PAD - src.shape[0]
  srcp = jnp.concatenate([src, jnp.zeros((pad,), jnp.int32)]).reshape(
      EDGE_BLOCKS, CHUNKS_PER_TILE, CHUNK)
  dstp = jnp.concatenate([dst, jnp.full((pad,), N, jnp.int32)]).reshape(
      EDGE_BLOCKS, CHUNKS_PER_TILE, CHUNK)
  zeros128 = jnp.zeros((ZROWS, D), jnp.float32)
  zeros64 = jnp.zeros((ZROWS, DC), jnp.float32)

  mp128 = _make_msgpass(D)
  mp64 = _make_msgpass(DC)

  a0p = mp128(srcp, dstp, zeros128, features)
  h1, ie0 = _tc_fuse1(a0p, W0, b0.reshape(1, D), Wi0, bi0.reshape(1, C))
  a1p = mp128(srcp, dstp, zeros128, h1)
  W2p = jnp.pad(W2, ((0, 0), (0, DC - C)))
  ie2, y2 = _tc_fuse2(a1p, W1, b1.reshape(1, D), Wi1, bi1.reshape(1, C),
                      W2p, b2.reshape(1, C), ie0)
  a2p = mp64(srcp, dstp, zeros64, y2)
  return _tc_fuse3(a2p, ie2)
